# Initial kernel scaffold; baseline (speedup 1.0000x reference)
import jax
import jax.numpy as jnp
from jax.experimental import pallas as pl


def kernel(inp1, inp2, embedding, W_ih_0_0, W_hh_0_0, b_ih_0_0, b_hh_0_0, W_ih_0_1, W_hh_0_1, b_ih_0_1, b_hh_0_1, W_ih_1_0, W_hh_1_0, b_ih_1_0, b_hh_1_0, W_ih_1_1, W_hh_1_1, b_ih_1_1, b_hh_1_1, Wy, Wh, w_att, fc_W, fc_b):
    raise NotImplementedError("write your pallas kernel here")


# speedup vs baseline: 7.5915x; 7.5915x over previous
import functools

import jax
import jax.numpy as jnp
from jax import lax
from jax.experimental import pallas as pl
from jax.experimental.pallas import tpu as pltpu
from jax.experimental.pallas import tpu_sc as plsc

S = 50
B = 128
BB = 2 * B
EMB = 128
HID = 128

_NC = 2
_NS = 16
_NW = _NC * _NS


def _sc_gather(table, idx):
    n = idx.shape[0]
    d = table.shape[1]
    per_w = n // _NW
    mesh = plsc.VectorSubcoreMesh(core_axis_name="c", subcore_axis_name="s")

    @functools.partial(
        pl.kernel,
        mesh=mesh,
        out_type=jax.ShapeDtypeStruct((n, d), jnp.float32),
        scratch_types=[
            pltpu.VMEM((per_w,), jnp.int32),
            pltpu.VMEM((per_w, d), jnp.float32),
            pltpu.SemaphoreType.DMA,
        ],
    )
    def gather_k(table_hbm, idx_hbm, out_hbm, idx_v, rows_v, sem):
        wid = lax.axis_index("s") * _NC + lax.axis_index("c")
        base = wid * per_w
        pltpu.sync_copy(idx_hbm.at[pl.ds(base, per_w)], idx_v)
        pltpu.async_copy(table_hbm.at[idx_v], rows_v, sem).wait()
        pltpu.sync_copy(rows_v, out_hbm.at[pl.ds(base, per_w)])

    return gather_k(table, idx)


def _tc_body(e_ref, wf1, wb1, uf1, ub1, bif1, bib1, bhf1, bhb1,
             wf2, wb2, uf2, ub2, bif2, bib2, bhf2, bhb2,
             wyt, wht, watt, fcw, fcb,
             out_ref, ys1, ys2, hf, hb, lg):
    f32 = jnp.float32
    H = HID

    def gates(gi, gh, h):
        r = jax.nn.sigmoid(gi[:, 0:H] + gh[:, 0:H])
        z = jax.nn.sigmoid(gi[:, H:2 * H] + gh[:, H:2 * H])
        n = jnp.tanh(gi[:, 2 * H:] + r * gh[:, 2 * H:])
        return (1.0 - z) * n + z * h

    def layer(x_ref, ys_ref, wf, wb, uf, ub, bif, bib, bhf, bhb):
        hf[...] = jnp.zeros((BB, H), f32)
        hb[...] = jnp.zeros((BB, H), f32)

        def step(t, _):
            x_f = x_ref[t]
            x_b = x_ref[S - 1 - t]
            h_f = hf[...]
            h_b = hb[...]
            gi_f = jnp.dot(x_f, wf[:], preferred_element_type=f32) + bif[:]
            gi_b = jnp.dot(x_b, wb[:], preferred_element_type=f32) + bib[:]
            gh_f = jnp.dot(h_f, uf[:], preferred_element_type=f32) + bhf[:]
            gh_b = jnp.dot(h_b, ub[:], preferred_element_type=f32) + bhb[:]
            h_f = gates(gi_f, gh_f, h_f)
            h_b = gates(gi_b, gh_b, h_b)
            hf[...] = h_f
            hb[...] = h_b
            ys_ref[t, :, 0:H] = h_f
            ys_ref[S - 1 - t, :, H:2 * H] = h_b
            return 0

        lax.fori_loop(0, S, step, 0)

    layer(e_ref, ys1, wf1, wb1, uf1, ub1, bif1, bib1, bhf1, bhb1)
    layer(ys1, ys2, wf2, wb2, uf2, ub2, bif2, bib2, bhf2, bhb2)

    pool = lax.fori_loop(
        0, S, lambda t, acc: acc + ys2[t], jnp.zeros((BB, 2 * H), f32))
    pool = pool * (1.0 / S)
    xh = jnp.dot(pool, wht[:], preferred_element_type=f32)
    wv = watt[:]

    def lstep(t, _):
        a = jnp.dot(ys2[t], wyt[:], preferred_element_type=f32)
        m = jnp.tanh(a + xh)
        lg[t] = jnp.sum(m * wv, axis=1)
        return 0

    lax.fori_loop(0, S, lstep, 0)

    logits = lg[...]
    mx = jnp.max(logits, axis=0)
    w = jnp.exp(logits - mx[None, :])
    sw = jnp.sum(w, axis=0)
    lg[...] = w

    racc = lax.fori_loop(
        0, S, lambda t, acc: acc + lg[t][:, None] * ys2[t],
        jnp.zeros((BB, 2 * H), f32))
    r = racc / sw[:, None]

    x1 = r[0:B]
    x2 = r[B:BB]
    x3 = x1 * x2
    x4 = jnp.abs(x1 - x2)
    xc = jnp.concatenate([x1, x3, x4, x2], axis=1)
    out_ref[...] = jnp.dot(xc, fcw[:], preferred_element_type=f32) + fcb[:]


_TC_SCRATCH = [
    pltpu.VMEM((S, BB, 2 * HID), jnp.float32),
    pltpu.VMEM((S, BB, 2 * HID), jnp.float32),
    pltpu.VMEM((BB, HID), jnp.float32),
    pltpu.VMEM((BB, HID), jnp.float32),
    pltpu.VMEM((S, BB), jnp.float32),
]

_TC_OUT = jax.ShapeDtypeStruct((B, 128), jnp.float32)


def kernel(inp1, inp2, embedding, W_ih_0_0, W_hh_0_0, b_ih_0_0, b_hh_0_0,
           W_ih_0_1, W_hh_0_1, b_ih_0_1, b_hh_0_1, W_ih_1_0, W_hh_1_0,
           b_ih_1_0, b_hh_1_0, W_ih_1_1, W_hh_1_1, b_ih_1_1, b_hh_1_1,
           Wy, Wh, w_att, fc_W, fc_b):
    idx = jnp.concatenate([inp1, inp2], axis=0).T.reshape(-1)
    e = _sc_gather(embedding, idx).reshape(S, BB, EMB)

    b2 = lambda v: v.reshape(1, -1)
    fcw = jnp.pad(fc_W.T, ((0, 0), (0, 128 - fc_W.shape[0])))
    fcb = jnp.pad(fc_b, (0, 128 - fc_b.shape[0])).reshape(1, -1)

    out = pl.pallas_call(
        _tc_body,
        out_shape=_TC_OUT,
        scratch_shapes=_TC_SCRATCH,
    )(e,
      W_ih_0_0.T, W_ih_0_1.T, W_hh_0_0.T, W_hh_0_1.T,
      b2(b_ih_0_0), b2(b_ih_0_1), b2(b_hh_0_0), b2(b_hh_0_1),
      W_ih_1_0.T, W_ih_1_1.T, W_hh_1_0.T, W_hh_1_1.T,
      b2(b_ih_1_0), b2(b_ih_1_1), b2(b_hh_1_0), b2(b_hh_1_1),
      Wy.T, Wh.T, w_att, fcw, fcb)
    return out[:, :fc_W.shape[0]]


# speedup vs baseline: 9.5203x; 1.2541x over previous
import functools

import jax
import jax.numpy as jnp
from jax import lax
from jax.experimental import pallas as pl
from jax.experimental.pallas import tpu as pltpu
from jax.experimental.pallas import tpu_sc as plsc

S = 50
B = 128
BB = 2 * B
EMB = 128
HID = 128
C = 10
NCH = S // C

_NC = 2
_NS = 16
_NW = _NC * _NS


def _sc_gather(table, idx):
    n = idx.shape[0]
    d = table.shape[1]
    per_w = n // _NW
    mesh = plsc.VectorSubcoreMesh(core_axis_name="c", subcore_axis_name="s")

    @functools.partial(
        pl.kernel,
        mesh=mesh,
        out_type=jax.ShapeDtypeStruct((n, d), jnp.float32),
        scratch_types=[
            pltpu.VMEM((per_w,), jnp.int32),
            pltpu.VMEM((per_w, d), jnp.float32),
            pltpu.SemaphoreType.DMA,
        ],
    )
    def gather_k(table_hbm, idx_hbm, out_hbm, idx_v, rows_v, sem):
        wid = lax.axis_index("s") * _NC + lax.axis_index("c")
        base = wid * per_w
        pltpu.sync_copy(idx_hbm.at[pl.ds(base, per_w)], idx_v)
        pltpu.async_copy(table_hbm.at[idx_v], rows_v, sem).wait()
        pltpu.sync_copy(rows_v, out_hbm.at[pl.ds(base, per_w)])

    return gather_k(table, idx)


def _tc_body(e_ref, wf1, wb1, uf1, ub1, bif1, bib1, bhf1, bhb1,
             wf2, wb2, uf2, ub2, bif2, bib2, bhf2, bhb2,
             wyt, wht, watt, fcw, fcb,
             out_ref, ys1, ys2, hf, hb, lg, gif, gib):
    f32 = jnp.float32
    H = HID

    def dot(a, b):
        return jnp.dot(a, b, preferred_element_type=f32)

    def gates(gi, gh, h):
        r = jax.nn.sigmoid(gi[:, 0:H] + gh[:, 0:H])
        z = jax.nn.sigmoid(gi[:, H:2 * H] + gh[:, H:2 * H])
        n = jnp.tanh(gi[:, 2 * H:] + r * gh[:, 2 * H:])
        return (1.0 - z) * n + z * h

    def layer(x_ref, in_dim, ys_ref, wf, wb, uf, ub, bif, bib, bhf, bhb):
        hf[...] = jnp.zeros((BB, H), f32)
        hb[...] = jnp.zeros((BB, H), f32)
        for c in range(NCH):
            xf = x_ref[pl.ds(c * C, C)]
            gif[...] = dot(xf.reshape(C * BB, in_dim), wf[:])
            xb = x_ref[pl.ds(S - (c + 1) * C, C)]
            gib[...] = dot(xb.reshape(C * BB, in_dim), wb[:])

            def step(t, _):
                tf = c * C + t
                h_f = hf[...]
                h_b = hb[...]
                gi_f = gif[pl.ds(t * BB, BB), :] + bif[:]
                gi_b = gib[pl.ds((C - 1 - t) * BB, BB), :] + bib[:]
                gh_f = dot(h_f, uf[:]) + bhf[:]
                gh_b = dot(h_b, ub[:]) + bhb[:]
                h_f = gates(gi_f, gh_f, h_f)
                h_b = gates(gi_b, gh_b, h_b)
                hf[...] = h_f
                hb[...] = h_b
                ys_ref[tf, :, 0:H] = h_f
                ys_ref[S - 1 - tf, :, H:2 * H] = h_b
                return 0

            lax.fori_loop(0, C, step, 0)

    layer(e_ref, EMB, ys1, wf1, wb1, uf1, ub1, bif1, bib1, bhf1, bhb1)
    layer(ys1, 2 * H, ys2, wf2, wb2, uf2, ub2, bif2, bib2, bhf2, bhb2)

    pacc = jnp.zeros((BB, 2 * H), f32)
    for c in range(NCH):
        pacc = pacc + jnp.sum(ys2[pl.ds(c * C, C)], axis=0)
    pool = pacc * (1.0 / S)
    xh = dot(pool, wht[:])
    wv = watt[:]

    for c in range(NCH):
        a = dot(ys2[pl.ds(c * C, C)].reshape(C * BB, 2 * H), wyt[:])
        m = jnp.tanh(a.reshape(C, BB, 2 * H) + xh[None])
        lg[pl.ds(c * C, C)] = jnp.sum(m * wv[None], axis=2)

    logits = lg[...]
    mx = jnp.max(logits, axis=0)
    w = jnp.exp(logits - mx[None, :])
    sw = jnp.sum(w, axis=0)

    racc = jnp.zeros((BB, 2 * H), f32)
    for c in range(NCH):
        wc = w[c * C:(c + 1) * C]
        racc = racc + jnp.sum(wc[:, :, None] * ys2[pl.ds(c * C, C)], axis=0)
    r = racc / sw[:, None]

    x1 = r[0:B]
    x2 = r[B:BB]
    x3 = x1 * x2
    x4 = jnp.abs(x1 - x2)
    xc = jnp.concatenate([x1, x3, x4, x2], axis=1)
    out_ref[...] = dot(xc, fcw[:]) + fcb[:]


_TC_SCRATCH = [
    pltpu.VMEM((S, BB, 2 * HID), jnp.float32),
    pltpu.VMEM((S, BB, 2 * HID), jnp.float32),
    pltpu.VMEM((BB, HID), jnp.float32),
    pltpu.VMEM((BB, HID), jnp.float32),
    pltpu.VMEM((S, BB), jnp.float32),
    pltpu.VMEM((C * BB, 3 * HID), jnp.float32),
    pltpu.VMEM((C * BB, 3 * HID), jnp.float32),
]

_TC_OUT = jax.ShapeDtypeStruct((B, 128), jnp.float32)


def kernel(inp1, inp2, embedding, W_ih_0_0, W_hh_0_0, b_ih_0_0, b_hh_0_0,
           W_ih_0_1, W_hh_0_1, b_ih_0_1, b_hh_0_1, W_ih_1_0, W_hh_1_0,
           b_ih_1_0, b_hh_1_0, W_ih_1_1, W_hh_1_1, b_ih_1_1, b_hh_1_1,
           Wy, Wh, w_att, fc_W, fc_b):
    idx = jnp.concatenate([inp1, inp2], axis=0).T.reshape(-1)
    e = _sc_gather(embedding, idx).reshape(S, BB, EMB)

    b2 = lambda v: v.reshape(1, -1)
    fcw = jnp.pad(fc_W.T, ((0, 0), (0, 128 - fc_W.shape[0])))
    fcb = jnp.pad(fc_b, (0, 128 - fc_b.shape[0])).reshape(1, -1)

    out = pl.pallas_call(
        _tc_body,
        out_shape=_TC_OUT,
        scratch_shapes=_TC_SCRATCH,
    )(e,
      W_ih_0_0.T, W_ih_0_1.T, W_hh_0_0.T, W_hh_0_1.T,
      b2(b_ih_0_0), b2(b_ih_0_1), b2(b_hh_0_0), b2(b_hh_0_1),
      W_ih_1_0.T, W_ih_1_1.T, W_hh_1_0.T, W_hh_1_1.T,
      b2(b_ih_1_0), b2(b_ih_1_1), b2(b_hh_1_0), b2(b_hh_1_1),
      Wy.T, Wh.T, w_att, fcw, fcb)
    return out[:, :fc_W.shape[0]]
